# Initial kernel scaffold; baseline (speedup 1.0000x reference)
import jax
import jax.numpy as jnp
from jax.experimental import pallas as pl


def kernel(x, params):
    raise NotImplementedError("write your pallas kernel here")


# speedup vs baseline: 1.2265x; 1.2265x over previous
import functools
import math

import jax
import jax.numpy as jnp
from jax.experimental import pallas as pl
from jax.experimental.pallas import tpu as pltpu

B, L0, C_IN = 2, 2048, 64
D_MODEL, N_HEADS, E_LAYERS, D_FF = 1024, 16, 3, 4096
FACTOR, PRED_LEN = 5, 96
DH = D_MODEL // N_HEADS

_F32 = jnp.float32


def _ln(x, g, b, eps=1e-5):
    m = jnp.mean(x, axis=-1, keepdims=True)
    v = jnp.mean((x - m) ** 2, axis=-1, keepdims=True)
    return (x - m) * jax.lax.rsqrt(v + eps) * g + b


def _embed_kern(xc_ref, w_ref, pos_ref, o_ref):
    o_ref[...] = (
        jnp.dot(xc_ref[...], w_ref[...], preferred_element_type=_F32)
        + pos_ref[...]
    )


def _embed(xcat, wcat, pos, bn=256):
    n, din = xcat.shape
    d = wcat.shape[1]
    nb_l = L0 // bn
    return pl.pallas_call(
        _embed_kern,
        grid=(n // bn,),
        in_specs=[
            pl.BlockSpec((bn, din), lambda i: (i, 0)),
            pl.BlockSpec((din, d), lambda i: (0, 0)),
            pl.BlockSpec((bn, d), lambda i: (i % nb_l, 0)),
        ],
        out_specs=pl.BlockSpec((bn, d), lambda i: (i, 0)),
        out_shape=jax.ShapeDtypeStruct((n, d), _F32),
    )(xcat, wcat, pos)


def _linear_kern(x_ref, w_ref, b_ref, o_ref):
    o_ref[...] = (
        jnp.dot(x_ref[...], w_ref[...], preferred_element_type=_F32)
        + b_ref[...]
    )


def _linear(x, w, b, bn=256):
    n, din = x.shape
    d = w.shape[1]
    return pl.pallas_call(
        _linear_kern,
        grid=(n // bn,),
        in_specs=[
            pl.BlockSpec((bn, din), lambda i: (i, 0)),
            pl.BlockSpec((din, d), lambda i: (0, 0)),
            pl.BlockSpec((1, d), lambda i: (0, 0)),
        ],
        out_specs=pl.BlockSpec((bn, d), lambda i: (i, 0)),
        out_shape=jax.ShapeDtypeStruct((n, d), _F32),
    )(x, w, b.reshape(1, d))


def _attn_kern(q_ref, kt_ref, v_ref, cnt_ref, o_ref, m_scr, *, ns, nsp, bq):
    lq = q_ref.shape[1]
    q = q_ref[0]
    kt = kt_ref[0]
    v = v_ref[0]

    def mblk(i, carry):
        qb = q_ref[0, pl.ds(i * bq, bq), :]
        cb = cnt_ref[pl.ds(i * bq, bq), :]
        s = jnp.dot(qb, kt, preferred_element_type=_F32)
        mean = jnp.sum(s * cb, axis=1) / ns
        mx = jnp.max(jnp.where(cb > 0.0, s, -jnp.inf), axis=1)
        m_scr[0, pl.ds(i * bq, bq)] = mx - mean
        return carry

    jax.lax.fori_loop(0, lq // bq, mblk, 0)
    m = m_scr[...]

    ji = jax.lax.broadcasted_iota(jnp.int32, (1, lq), 1)

    def sel(t, carry):
        m, tc, tr = carry
        mval = jnp.max(m)
        it = jnp.min(jnp.where(m >= mval, ji, lq)).astype(jnp.int32)
        tc = jnp.where(
            jax.lax.broadcasted_iota(jnp.int32, (nsp, 1), 0) == t, it, tc
        )
        tr = jnp.where(
            jax.lax.broadcasted_iota(jnp.int32, (1, nsp), 1) == t, it, tr
        )
        m = jnp.where(ji == it, -jnp.inf, m)
        return m, tc, tr

    _, tops_c, tops_r = jax.lax.fori_loop(
        0,
        ns,
        sel,
        (m, jnp.full((nsp, 1), lq, jnp.int32), jnp.full((1, nsp), lq, jnp.int32)),
    )

    oh = (tops_c == ji).astype(_F32)
    il = jax.lax.broadcasted_iota(jnp.int32, (lq, 1), 0)
    oht = (il == tops_r).astype(_F32)

    qred = jnp.dot(oh, q, preferred_element_type=_F32)
    scores = jnp.dot(qred, kt, preferred_element_type=_F32) * (
        1.0 / math.sqrt(DH)
    )
    smax = jnp.max(scores, axis=1, keepdims=True)
    e = jnp.exp(scores - smax)
    attn = e / jnp.sum(e, axis=1, keepdims=True)
    ctx = jnp.dot(attn, v, preferred_element_type=_F32)

    vmean = jnp.mean(v, axis=0, keepdims=True)
    contrib = jnp.dot(oht, ctx, preferred_element_type=_F32)
    sel_ind = jnp.sum(oht, axis=1, keepdims=True)
    o_ref[0] = jnp.where(sel_ind > 0.0, contrib, vmean)


def _attention(q, kt, v, cnt, ns, bq=256):
    bh, lq, dh = q.shape
    nsp = 64
    kern = functools.partial(_attn_kern, ns=ns, nsp=nsp, bq=min(bq, lq))
    return pl.pallas_call(
        kern,
        grid=(bh,),
        in_specs=[
            pl.BlockSpec((1, lq, dh), lambda i: (i, 0, 0)),
            pl.BlockSpec((1, dh, lq), lambda i: (i, 0, 0)),
            pl.BlockSpec((1, lq, dh), lambda i: (i, 0, 0)),
            pl.BlockSpec((lq, lq), lambda i: (0, 0)),
        ],
        out_specs=pl.BlockSpec((1, lq, dh), lambda i: (i, 0, 0)),
        out_shape=jax.ShapeDtypeStruct((bh, lq, dh), _F32),
        scratch_shapes=[pltpu.VMEM((1, lq), _F32)],
    )(q, kt, v, cnt)


def _ores_kern(ctx_ref, x_ref, w_ref, b_ref, g_ref, bb_ref, xres_ref, y_ref):
    xr = (
        x_ref[...]
        + jnp.dot(ctx_ref[...], w_ref[...], preferred_element_type=_F32)
        + b_ref[...]
    )
    xres_ref[...] = xr
    y_ref[...] = _ln(xr, g_ref[...], bb_ref[...])


def _ores(ctx, x, w, b, g, bb, bn=256):
    n, d = x.shape
    return pl.pallas_call(
        _ores_kern,
        grid=(n // bn,),
        in_specs=[
            pl.BlockSpec((bn, d), lambda i: (i, 0)),
            pl.BlockSpec((bn, d), lambda i: (i, 0)),
            pl.BlockSpec((d, d), lambda i: (0, 0)),
            pl.BlockSpec((1, d), lambda i: (0, 0)),
            pl.BlockSpec((1, d), lambda i: (0, 0)),
            pl.BlockSpec((1, d), lambda i: (0, 0)),
        ],
        out_specs=[
            pl.BlockSpec((bn, d), lambda i: (i, 0)),
            pl.BlockSpec((bn, d), lambda i: (i, 0)),
        ],
        out_shape=[
            jax.ShapeDtypeStruct((n, d), _F32),
            jax.ShapeDtypeStruct((n, d), _F32),
        ],
    )(ctx, x, w, b.reshape(1, d), g.reshape(1, d), bb.reshape(1, d))


def _ffn_kern(y_ref, xr_ref, w1_ref, b1_ref, w2_ref, b2_ref, g_ref, bb_ref, o_ref):
    h = jnp.dot(y_ref[...], w1_ref[...], preferred_element_type=_F32) + b1_ref[...]
    h = 0.5 * h * (1.0 + jax.lax.erf(h * (1.0 / math.sqrt(2.0))))
    y2 = jnp.dot(h, w2_ref[...], preferred_element_type=_F32) + b2_ref[...]
    o_ref[...] = _ln(xr_ref[...] + y2, g_ref[...], bb_ref[...])


def _ffn(y, xres, w1, b1, w2, b2, g, bb, bn=256):
    n, d = y.shape
    dff = w1.shape[1]
    return pl.pallas_call(
        _ffn_kern,
        grid=(n // bn,),
        in_specs=[
            pl.BlockSpec((bn, d), lambda i: (i, 0)),
            pl.BlockSpec((bn, d), lambda i: (i, 0)),
            pl.BlockSpec((d, dff), lambda i: (0, 0)),
            pl.BlockSpec((1, dff), lambda i: (0, 0)),
            pl.BlockSpec((dff, d), lambda i: (0, 0)),
            pl.BlockSpec((1, d), lambda i: (0, 0)),
            pl.BlockSpec((1, d), lambda i: (0, 0)),
            pl.BlockSpec((1, d), lambda i: (0, 0)),
        ],
        out_specs=pl.BlockSpec((bn, d), lambda i: (i, 0)),
        out_shape=jax.ShapeDtypeStruct((n, d), _F32),
    )(
        y,
        xres,
        w1,
        b1.reshape(1, dff),
        w2,
        b2.reshape(1, d),
        g.reshape(1, d),
        bb.reshape(1, d),
    )


def _distill_kern(x_ref, w_ref, b_ref, g_ref, bb_ref, o_ref, *, bl, nblk):
    j = pl.program_id(1)
    d = x_ref.shape[2]
    xb = x_ref[0, pl.ds(j * bl, bl + 4), :]
    y = (
        jnp.dot(xb[: bl + 2], w_ref[0], preferred_element_type=_F32)
        + jnp.dot(xb[1 : bl + 3], w_ref[1], preferred_element_type=_F32)
        + jnp.dot(xb[2 : bl + 4], w_ref[2], preferred_element_type=_F32)
        + b_ref[...]
    )
    y = y * (1.0 / math.sqrt(1.0 + 1e-5)) * g_ref[...] + bb_ref[...]
    y = jnp.where(y > 0.0, y, jnp.exp(y) - 1.0)
    ri = jax.lax.broadcasted_iota(jnp.int32, (bl + 2, 1), 0)
    edge = ((ri == 0) & (j == 0)) | ((ri == bl + 1) & (j == nblk - 1))
    y = jnp.where(edge, -jnp.inf, y)
    o_ref[0] = jnp.maximum(jnp.maximum(y[:bl], y[1 : bl + 1]), y[2 : bl + 2])


def _distill(x, w, b, g, bb, bl=256):
    bsz, l, d = x.shape
    xext = jnp.concatenate(
        [x[:, -2:, :], x, x[:, :2, :], jnp.zeros((bsz, 4, d), _F32)], axis=1
    )
    nblk = l // bl
    kern = functools.partial(_distill_kern, bl=bl, nblk=nblk)
    z = pl.pallas_call(
        kern,
        grid=(bsz, nblk),
        in_specs=[
            pl.BlockSpec((1, l + 8, d), lambda i, j: (i, 0, 0)),
            pl.BlockSpec((3, d, d), lambda i, j: (0, 0, 0)),
            pl.BlockSpec((1, d), lambda i, j: (0, 0)),
            pl.BlockSpec((1, d), lambda i, j: (0, 0)),
            pl.BlockSpec((1, d), lambda i, j: (0, 0)),
        ],
        out_specs=pl.BlockSpec((1, bl, d), lambda i, j: (i, j, 0)),
        out_shape=jax.ShapeDtypeStruct((bsz, l, d), _F32),
    )(xext, w, b.reshape(1, d), g.reshape(1, d), bb.reshape(1, d))
    return z[:, ::2, :]


def _head_kern(h_ref, x_ref, g_ref, b_ref, hw_ref, hb_ref, sw_ref, sb_ref, o_ref):
    for bi in range(B):
        h = _ln(h_ref[bi], g_ref[...], b_ref[...])
        pooled = jnp.mean(h, axis=0, keepdims=True)
        xm = jnp.mean(x_ref[bi], axis=0, keepdims=True)
        o_ref[pl.ds(bi, 1), :] = (
            jnp.dot(pooled, hw_ref[...], preferred_element_type=_F32)
            + hb_ref[...]
            + jnp.dot(xm, sw_ref[...], preferred_element_type=_F32)
            + sb_ref[...]
        )


def _head(h, x, g, b, hw, hb, sw, sb):
    _, l2, d = h.shape
    return pl.pallas_call(
        _head_kern,
        out_shape=jax.ShapeDtypeStruct((B, PRED_LEN), _F32),
    )(
        h,
        x,
        g.reshape(1, d),
        b.reshape(1, d),
        hw,
        hb.reshape(1, PRED_LEN),
        sw,
        sb.reshape(1, PRED_LEN),
    )


def _pos_emb(l, d):
    position = jnp.arange(l, dtype=_F32)[:, None]
    div = jnp.exp(
        jnp.arange(0, d, 2, dtype=_F32) * (-math.log(10000.0) / d)
    )
    pe = jnp.zeros((l, d), _F32)
    pe = pe.at[:, 0::2].set(jnp.sin(position * div))
    pe = pe.at[:, 1::2].set(jnp.cos(position * div))
    return pe


def _sample_consts(layer_i, lq):
    sample_k = min(lq, int(FACTOR * math.log(lq + 1)))
    n_top = min(lq, int(FACTOR * math.log(lq + 1)))
    key = jax.random.fold_in(jax.random.key(42), layer_i)
    idx = jax.random.randint(key, (lq, sample_k), 0, lq)
    cnt = jnp.zeros((lq, lq), _F32).at[jnp.arange(lq)[:, None], idx].add(1.0)
    return cnt, n_top


def kernel(x, params):
    p = params
    bsz, l, _ = x.shape

    xcat = jnp.concatenate(
        [jnp.roll(x, 1, axis=1), x, jnp.roll(x, -1, axis=1)], axis=-1
    ).reshape(bsz * l, 3 * C_IN)
    wcat = p["token_conv_w"].reshape(3 * C_IN, D_MODEL)
    h = _embed(xcat, wcat, _pos_emb(l, D_MODEL))

    lq = l
    for i in range(E_LAYERS):
        lp = p["layers"][i]
        cnt, n_top = _sample_consts(i, lq)
        wqkv = jnp.concatenate([lp["q_w"], lp["k_w"], lp["v_w"]], axis=1)
        bqkv = jnp.concatenate([lp["q_b"], lp["k_b"], lp["v_b"]], axis=0)
        qkv = _linear(h, wqkv, bqkv)
        qkv = qkv.reshape(bsz, lq, 3, N_HEADS, DH)
        q = qkv[:, :, 0].transpose(0, 2, 1, 3).reshape(bsz * N_HEADS, lq, DH)
        kt = qkv[:, :, 1].transpose(0, 2, 3, 1).reshape(bsz * N_HEADS, DH, lq)
        v = qkv[:, :, 2].transpose(0, 2, 1, 3).reshape(bsz * N_HEADS, lq, DH)
        ctx = _attention(q, kt, v, cnt, n_top)
        ctx = (
            ctx.reshape(bsz, N_HEADS, lq, DH)
            .transpose(0, 2, 1, 3)
            .reshape(bsz * lq, D_MODEL)
        )
        xres, y = _ores(ctx, h, lp["o_w"], lp["o_b"], lp["n1_g"], lp["n1_b"])
        h = _ffn(
            y, xres, lp["ff1_w"], lp["ff1_b"], lp["ff2_w"], lp["ff2_b"],
            lp["n2_g"], lp["n2_b"],
        )
        if i < E_LAYERS - 1:
            cp = p["convs"][i]
            h3 = h.reshape(bsz, lq, D_MODEL)
            h3 = _distill(h3, cp["conv_w"], cp["conv_b"], cp["bn_g"], cp["bn_b"])
            lq = lq // 2
            h = h3.reshape(bsz * lq, D_MODEL)

    h3 = h.reshape(bsz, lq, D_MODEL)
    return _head(
        h3, x, p["norm_g"], p["norm_b"], p["head_w"], p["head_b"],
        p["skip_w"], p["skip_b"],
    )


# speedup vs baseline: 1.8707x; 1.5253x over previous
import functools
import math

import jax
import jax.numpy as jnp
from jax.experimental import pallas as pl
from jax.experimental.pallas import tpu as pltpu

B, L0, C_IN = 2, 2048, 64
D_MODEL, N_HEADS, E_LAYERS, D_FF = 1024, 16, 3, 4096
FACTOR, PRED_LEN = 5, 96
DH = D_MODEL // N_HEADS

_F32 = jnp.float32


def _ln(x, g, b, eps=1e-5):
    m = jnp.mean(x, axis=-1, keepdims=True)
    v = jnp.mean((x - m) ** 2, axis=-1, keepdims=True)
    return (x - m) * jax.lax.rsqrt(v + eps) * g + b


def _embed_kern(xc_ref, w_ref, pos_ref, o_ref):
    o_ref[...] = (
        jnp.dot(xc_ref[...], w_ref[...], preferred_element_type=_F32)
        + pos_ref[...]
    )


def _embed(xcat, wcat, pos, bn=256):
    n, din = xcat.shape
    d = wcat.shape[1]
    nb_l = L0 // bn
    return pl.pallas_call(
        _embed_kern,
        grid=(n // bn,),
        in_specs=[
            pl.BlockSpec((bn, din), lambda i: (i, 0)),
            pl.BlockSpec((din, d), lambda i: (0, 0)),
            pl.BlockSpec((bn, d), lambda i: (i % nb_l, 0)),
        ],
        out_specs=pl.BlockSpec((bn, d), lambda i: (i, 0)),
        out_shape=jax.ShapeDtypeStruct((n, d), _F32),
    )(xcat, wcat, pos)


def _linear_kern(x_ref, w_ref, b_ref, o_ref):
    o_ref[...] = (
        jnp.dot(x_ref[...], w_ref[...], preferred_element_type=_F32)
        + b_ref[...]
    )


def _linear(x, w, b, bn=256):
    n, din = x.shape
    d = w.shape[1]
    return pl.pallas_call(
        _linear_kern,
        grid=(n // bn,),
        in_specs=[
            pl.BlockSpec((bn, din), lambda i: (i, 0)),
            pl.BlockSpec((din, d), lambda i: (0, 0)),
            pl.BlockSpec((1, d), lambda i: (0, 0)),
        ],
        out_specs=pl.BlockSpec((bn, d), lambda i: (i, 0)),
        out_shape=jax.ShapeDtypeStruct((n, d), _F32),
    )(x, w, b.reshape(1, d))


def _measure_kern(q_ref, kt_ref, cnt_ref, m_ref, *, ns, bq):
    lq = q_ref.shape[1]
    kt = kt_ref[0]

    def mblk(i, carry):
        qb = q_ref[0, pl.ds(i * bq, bq), :]
        cb = cnt_ref[pl.ds(i * bq, bq), :]
        s = jnp.dot(qb, kt, preferred_element_type=_F32)
        mean = jnp.sum(s * cb, axis=1) / ns
        mx = jnp.max(jnp.where(cb > 0.0, s, -jnp.inf), axis=1)
        m_ref[0, 0, pl.ds(i * bq, bq)] = mx - mean
        return carry

    jax.lax.fori_loop(0, lq // bq, mblk, 0)


def _topk_kern(m_ref, t_ref, *, ns, nsp):
    bh, _, lq = m_ref.shape
    m = m_ref[:, 0, :]
    ji = jax.lax.broadcasted_iota(jnp.int32, (bh, lq), 1)
    ci = jax.lax.broadcasted_iota(jnp.int32, (bh, nsp), 1)

    def sel(t, carry):
        m, tops = carry
        mval = jnp.max(m, axis=1, keepdims=True)
        it = jnp.min(jnp.where(m >= mval, ji, lq), axis=1, keepdims=True)
        tops = jnp.where(ci == t, it, tops)
        m = jnp.where(ji == it, -jnp.inf, m)
        return m, tops

    _, tops = jax.lax.fori_loop(
        0, ns, sel, (m, jnp.full((bh, nsp), lq, jnp.int32))
    )
    t_ref[:, 0, :] = tops


def _ctx_kern(q_ref, kt_ref, v_ref, t_ref, o_ref, *, ns):
    lq = q_ref.shape[1]
    kt = kt_ref[0]
    v = v_ref[0]
    rows = [q_ref[0, pl.ds(t_ref[0, 0, n], 1), :] for n in range(ns)]
    qred = jnp.concatenate(rows, axis=0)
    scores = jnp.dot(qred, kt, preferred_element_type=_F32) * (
        1.0 / math.sqrt(DH)
    )
    smax = jnp.max(scores, axis=1, keepdims=True)
    e = jnp.exp(scores - smax)
    attn = e / jnp.sum(e, axis=1, keepdims=True)
    ctx = jnp.dot(attn, v, preferred_element_type=_F32)
    o_ref[0] = jnp.broadcast_to(jnp.mean(v, axis=0, keepdims=True), v.shape)
    for n in range(ns):
        o_ref[0, pl.ds(t_ref[0, 0, n], 1), :] = ctx[n : n + 1, :]


def _attention(q, kt, v, cnt, ns, bq=256):
    bh, lq, dh = q.shape
    nsp = 64
    m = pl.pallas_call(
        functools.partial(_measure_kern, ns=ns, bq=min(bq, lq)),
        grid=(bh,),
        in_specs=[
            pl.BlockSpec((1, lq, dh), lambda i: (i, 0, 0)),
            pl.BlockSpec((1, dh, lq), lambda i: (i, 0, 0)),
            pl.BlockSpec((lq, lq), lambda i: (0, 0)),
        ],
        out_specs=pl.BlockSpec((1, 1, lq), lambda i: (i, 0, 0)),
        out_shape=jax.ShapeDtypeStruct((bh, 1, lq), _F32),
    )(q, kt, cnt)
    tops = pl.pallas_call(
        functools.partial(_topk_kern, ns=ns, nsp=nsp),
        out_shape=jax.ShapeDtypeStruct((bh, 1, nsp), jnp.int32),
    )(m)
    return pl.pallas_call(
        functools.partial(_ctx_kern, ns=ns),
        grid=(bh,),
        in_specs=[
            pl.BlockSpec((1, lq, dh), lambda i: (i, 0, 0)),
            pl.BlockSpec((1, dh, lq), lambda i: (i, 0, 0)),
            pl.BlockSpec((1, lq, dh), lambda i: (i, 0, 0)),
            pl.BlockSpec((1, 1, nsp), lambda i: (i, 0, 0), memory_space=pltpu.SMEM),
        ],
        out_specs=pl.BlockSpec((1, lq, dh), lambda i: (i, 0, 0)),
        out_shape=jax.ShapeDtypeStruct((bh, lq, dh), _F32),
    )(q, kt, v, tops)


def _ores_kern(ctx_ref, x_ref, w_ref, b_ref, g_ref, bb_ref, xres_ref, y_ref):
    xr = (
        x_ref[...]
        + jnp.dot(ctx_ref[...], w_ref[...], preferred_element_type=_F32)
        + b_ref[...]
    )
    xres_ref[...] = xr
    y_ref[...] = _ln(xr, g_ref[...], bb_ref[...])


def _ores(ctx, x, w, b, g, bb, bn=256):
    n, d = x.shape
    return pl.pallas_call(
        _ores_kern,
        grid=(n // bn,),
        in_specs=[
            pl.BlockSpec((bn, d), lambda i: (i, 0)),
            pl.BlockSpec((bn, d), lambda i: (i, 0)),
            pl.BlockSpec((d, d), lambda i: (0, 0)),
            pl.BlockSpec((1, d), lambda i: (0, 0)),
            pl.BlockSpec((1, d), lambda i: (0, 0)),
            pl.BlockSpec((1, d), lambda i: (0, 0)),
        ],
        out_specs=[
            pl.BlockSpec((bn, d), lambda i: (i, 0)),
            pl.BlockSpec((bn, d), lambda i: (i, 0)),
        ],
        out_shape=[
            jax.ShapeDtypeStruct((n, d), _F32),
            jax.ShapeDtypeStruct((n, d), _F32),
        ],
    )(ctx, x, w, b.reshape(1, d), g.reshape(1, d), bb.reshape(1, d))


def _ffn_kern(y_ref, xr_ref, w1_ref, b1_ref, w2_ref, b2_ref, g_ref, bb_ref, o_ref):
    h = jnp.dot(y_ref[...], w1_ref[...], preferred_element_type=_F32) + b1_ref[...]
    h = 0.5 * h * (1.0 + jax.lax.erf(h * (1.0 / math.sqrt(2.0))))
    y2 = jnp.dot(h, w2_ref[...], preferred_element_type=_F32) + b2_ref[...]
    o_ref[...] = _ln(xr_ref[...] + y2, g_ref[...], bb_ref[...])


def _ffn(y, xres, w1, b1, w2, b2, g, bb, bn=256):
    n, d = y.shape
    dff = w1.shape[1]
    return pl.pallas_call(
        _ffn_kern,
        grid=(n // bn,),
        in_specs=[
            pl.BlockSpec((bn, d), lambda i: (i, 0)),
            pl.BlockSpec((bn, d), lambda i: (i, 0)),
            pl.BlockSpec((d, dff), lambda i: (0, 0)),
            pl.BlockSpec((1, dff), lambda i: (0, 0)),
            pl.BlockSpec((dff, d), lambda i: (0, 0)),
            pl.BlockSpec((1, d), lambda i: (0, 0)),
            pl.BlockSpec((1, d), lambda i: (0, 0)),
            pl.BlockSpec((1, d), lambda i: (0, 0)),
        ],
        out_specs=pl.BlockSpec((bn, d), lambda i: (i, 0)),
        out_shape=jax.ShapeDtypeStruct((n, d), _F32),
    )(
        y,
        xres,
        w1,
        b1.reshape(1, dff),
        w2,
        b2.reshape(1, d),
        g.reshape(1, d),
        bb.reshape(1, d),
    )


def _distill_kern(x_ref, w_ref, b_ref, g_ref, bb_ref, o_ref, *, bl, nblk):
    j = pl.program_id(1)
    d = x_ref.shape[2]
    xb = x_ref[0, pl.ds(j * bl, bl + 4), :]
    y = (
        jnp.dot(xb[: bl + 2], w_ref[0], preferred_element_type=_F32)
        + jnp.dot(xb[1 : bl + 3], w_ref[1], preferred_element_type=_F32)
        + jnp.dot(xb[2 : bl + 4], w_ref[2], preferred_element_type=_F32)
        + b_ref[...]
    )
    y = y * (1.0 / math.sqrt(1.0 + 1e-5)) * g_ref[...] + bb_ref[...]
    y = jnp.where(y > 0.0, y, jnp.exp(y) - 1.0)
    ri = jax.lax.broadcasted_iota(jnp.int32, (bl + 2, 1), 0)
    edge = ((ri == 0) & (j == 0)) | ((ri == bl + 1) & (j == nblk - 1))
    y = jnp.where(edge, -jnp.inf, y)
    o_ref[0] = jnp.maximum(jnp.maximum(y[:bl], y[1 : bl + 1]), y[2 : bl + 2])


def _distill(x, w, b, g, bb, bl=256):
    bsz, l, d = x.shape
    xext = jnp.concatenate(
        [x[:, -2:, :], x, x[:, :2, :], jnp.zeros((bsz, 4, d), _F32)], axis=1
    )
    nblk = l // bl
    kern = functools.partial(_distill_kern, bl=bl, nblk=nblk)
    z = pl.pallas_call(
        kern,
        grid=(bsz, nblk),
        in_specs=[
            pl.BlockSpec((1, l + 8, d), lambda i, j: (i, 0, 0)),
            pl.BlockSpec((3, d, d), lambda i, j: (0, 0, 0)),
            pl.BlockSpec((1, d), lambda i, j: (0, 0)),
            pl.BlockSpec((1, d), lambda i, j: (0, 0)),
            pl.BlockSpec((1, d), lambda i, j: (0, 0)),
        ],
        out_specs=pl.BlockSpec((1, bl, d), lambda i, j: (i, j, 0)),
        out_shape=jax.ShapeDtypeStruct((bsz, l, d), _F32),
    )(xext, w, b.reshape(1, d), g.reshape(1, d), bb.reshape(1, d))
    return z[:, ::2, :]


def _head_kern(h_ref, x_ref, g_ref, b_ref, hw_ref, hb_ref, sw_ref, sb_ref, o_ref):
    for bi in range(B):
        h = _ln(h_ref[bi], g_ref[...], b_ref[...])
        pooled = jnp.mean(h, axis=0, keepdims=True)
        xm = jnp.mean(x_ref[bi], axis=0, keepdims=True)
        o_ref[pl.ds(bi, 1), :] = (
            jnp.dot(pooled, hw_ref[...], preferred_element_type=_F32)
            + hb_ref[...]
            + jnp.dot(xm, sw_ref[...], preferred_element_type=_F32)
            + sb_ref[...]
        )


def _head(h, x, g, b, hw, hb, sw, sb):
    _, l2, d = h.shape
    return pl.pallas_call(
        _head_kern,
        out_shape=jax.ShapeDtypeStruct((B, PRED_LEN), _F32),
    )(
        h,
        x,
        g.reshape(1, d),
        b.reshape(1, d),
        hw,
        hb.reshape(1, PRED_LEN),
        sw,
        sb.reshape(1, PRED_LEN),
    )


def _pos_emb(l, d):
    position = jnp.arange(l, dtype=_F32)[:, None]
    div = jnp.exp(
        jnp.arange(0, d, 2, dtype=_F32) * (-math.log(10000.0) / d)
    )
    pe = jnp.zeros((l, d), _F32)
    pe = pe.at[:, 0::2].set(jnp.sin(position * div))
    pe = pe.at[:, 1::2].set(jnp.cos(position * div))
    return pe


def _sample_consts(layer_i, lq):
    sample_k = min(lq, int(FACTOR * math.log(lq + 1)))
    n_top = min(lq, int(FACTOR * math.log(lq + 1)))
    key = jax.random.fold_in(jax.random.key(42), layer_i)
    idx = jax.random.randint(key, (lq, sample_k), 0, lq)
    cnt = jnp.zeros((lq, lq), _F32).at[jnp.arange(lq)[:, None], idx].add(1.0)
    return cnt, n_top


def kernel(x, params):
    p = params
    bsz, l, _ = x.shape

    xcat = jnp.concatenate(
        [jnp.roll(x, 1, axis=1), x, jnp.roll(x, -1, axis=1)], axis=-1
    ).reshape(bsz * l, 3 * C_IN)
    wcat = p["token_conv_w"].reshape(3 * C_IN, D_MODEL)
    h = _embed(xcat, wcat, _pos_emb(l, D_MODEL))

    lq = l
    for i in range(E_LAYERS):
        lp = p["layers"][i]
        cnt, n_top = _sample_consts(i, lq)
        wqkv = jnp.concatenate([lp["q_w"], lp["k_w"], lp["v_w"]], axis=1)
        bqkv = jnp.concatenate([lp["q_b"], lp["k_b"], lp["v_b"]], axis=0)
        qkv = _linear(h, wqkv, bqkv)
        qkv = qkv.reshape(bsz, lq, 3, N_HEADS, DH)
        q = qkv[:, :, 0].transpose(0, 2, 1, 3).reshape(bsz * N_HEADS, lq, DH)
        kt = qkv[:, :, 1].transpose(0, 2, 3, 1).reshape(bsz * N_HEADS, DH, lq)
        v = qkv[:, :, 2].transpose(0, 2, 1, 3).reshape(bsz * N_HEADS, lq, DH)
        ctx = _attention(q, kt, v, cnt, n_top)
        ctx = (
            ctx.reshape(bsz, N_HEADS, lq, DH)
            .transpose(0, 2, 1, 3)
            .reshape(bsz * lq, D_MODEL)
        )
        xres, y = _ores(ctx, h, lp["o_w"], lp["o_b"], lp["n1_g"], lp["n1_b"])
        h = _ffn(
            y, xres, lp["ff1_w"], lp["ff1_b"], lp["ff2_w"], lp["ff2_b"],
            lp["n2_g"], lp["n2_b"],
        )
        if i < E_LAYERS - 1:
            cp = p["convs"][i]
            h3 = h.reshape(bsz, lq, D_MODEL)
            h3 = _distill(h3, cp["conv_w"], cp["conv_b"], cp["bn_g"], cp["bn_b"])
            lq = lq // 2
            h = h3.reshape(bsz * lq, D_MODEL)

    h3 = h.reshape(bsz, lq, D_MODEL)
    return _head(
        h3, x, p["norm_g"], p["norm_b"], p["head_w"], p["head_b"],
        p["skip_w"], p["skip_b"],
    )
